# Initial kernel scaffold; baseline (speedup 1.0000x reference)
import jax
import jax.numpy as jnp
from jax.experimental import pallas as pl


def kernel(video, audio, title, edge_index, user_emb, Wf, bf, W0l, W0r, b0, g0, be0, W1l, W1r, b1, g1, be1):
    raise NotImplementedError("write your pallas kernel here")


# speedup vs baseline: 3.9229x; 3.9229x over previous
import functools

import jax
import jax.numpy as jnp
from jax import lax
from jax.experimental import pallas as pl
from jax.experimental.pallas import tpu as pltpu
from jax.experimental.pallas import tpu_sc as plsc

_USER = 10000
_ITEM = 40000
_N = _USER + _ITEM
_E = 800000
_D = 64
_HID = 128

_NC = 2
_NS = 16
_HALF = _N // _NC
_DUMP_MASK = 1023
_ACC_ROWS = 26112
_RPT = _ACC_ROWS // _NS
_EB = 128
_NBLK = _E // _EB
_BQ, _BR = _NBLK // _NS, _NBLK % _NS


@functools.lru_cache(maxsize=None)
def _make_segsum(with_deg: bool):
  mesh = plsc.VectorSubcoreMesh(core_axis_name="c", subcore_axis_name="s",
                                num_cores=_NC, num_subcores=_NS)
  out_type = [jax.ShapeDtypeStruct((_NC * _ACC_ROWS, _D), jnp.float32)]
  scratch = [
      pltpu.VMEM((_EB,), jnp.int32),
      pltpu.VMEM((_EB,), jnp.int32),
      pltpu.VMEM((_EB, _D), jnp.float32),
      pltpu.VMEM((128, _D), jnp.float32),
      pltpu.VMEM_SHARED((_ACC_ROWS, _D), jnp.float32),
  ]
  if with_deg:
    out_type.append(jax.ShapeDtypeStruct((_NC * _ACC_ROWS,), jnp.float32))
    scratch += [
        pltpu.VMEM((_EB,), jnp.float32),
        pltpu.VMEM((_RPT,), jnp.float32),
        pltpu.VMEM_SHARED((_ACC_ROWS,), jnp.float32),
    ]

  @functools.partial(
      pl.kernel, mesh=mesh, out_type=tuple(out_type), scratch_types=scratch,
      compiler_params=pltpu.CompilerParams(use_tc_tiling_on_sc=False))
  def segsum(x_hbm, src_hbm, dst_hbm, *refs):
    if with_deg:
      (agg_out, deg_out, src_st, dst_st, rows_v, zrow_v, acc_sh,
       ones_v, zdeg_v, deg_sh) = refs
    else:
      agg_out, src_st, dst_st, rows_v, zrow_v, acc_sh = refs
    c = lax.axis_index("c")
    s = lax.axis_index("s")
    lo = c * _HALF
    hi = lo + _HALF
    row0 = s * _RPT
    z16 = jnp.zeros((16,), jnp.float32)
    lane = lax.iota(jnp.int32, 16)

    def zrow_body(r, _):
      for j in range(_D // 16):
        zrow_v[r, pl.ds(j * 16, 16)] = z16
      return 0
    lax.fori_loop(0, 128, zrow_body, 0)
    for k in range(_RPT // 128):
      pltpu.sync_copy(zrow_v, acc_sh.at[pl.ds(row0 + k * 128, 128)])
    rem = _RPT % 128
    if rem:
      pltpu.sync_copy(zrow_v.at[pl.ds(0, rem)],
                      acc_sh.at[pl.ds(row0 + _RPT - rem, rem)])
    if with_deg:
      def zdeg_body(r, _):
        zdeg_v[pl.ds(r * 16, 16)] = z16
        ones_v[pl.ds((r % 8) * 16, 16)] = z16 + 1.0
        return 0
      lax.fori_loop(0, _RPT // 16, zdeg_body, 0)
      pltpu.sync_copy(zdeg_v, deg_sh.at[pl.ds(row0, _RPT)])
    plsc.subcore_barrier()

    blk0 = s * _BQ + jnp.minimum(s, _BR)
    nblk = _BQ + jnp.where(s < _BR, 1, 0)

    def body(i, _):
      e0 = (blk0 + i) * _EB
      pltpu.sync_copy(src_hbm.at[pl.ds(e0, _EB)], src_st)
      pltpu.sync_copy(dst_hbm.at[pl.ds(e0, _EB)], dst_st)
      for j in range(_EB // 16):
        d = dst_st[pl.ds(j * 16, 16)]
        mine = (d >= lo) & (d < hi)
        dump = _HALF + ((i * _EB + j * 16 + lane) & _DUMP_MASK)
        dst_st[pl.ds(j * 16, 16)] = jnp.where(mine, d - lo, dump)
      pltpu.sync_copy(x_hbm.at[src_st], rows_v)
      pltpu.sync_copy(rows_v, acc_sh.at[dst_st], add=True)
      if with_deg:
        pltpu.sync_copy(ones_v, deg_sh.at[dst_st], add=True)
      return 0
    lax.fori_loop(0, nblk, body, 0)
    plsc.subcore_barrier()

    o0 = c * _ACC_ROWS + row0
    pltpu.sync_copy(acc_sh.at[pl.ds(row0, _RPT)], agg_out.at[pl.ds(o0, _RPT)])
    if with_deg:
      pltpu.sync_copy(deg_sh.at[pl.ds(row0, _RPT)], deg_out.at[pl.ds(o0, _RPT)])

  return segsum


def _unpad(a):
  return jnp.concatenate([a[:_HALF], a[_ACC_ROWS:_ACC_ROWS + _HALF]], axis=0)


_BM = 2000


def _full(shape):
  return pl.BlockSpec(shape, lambda i: (0, 0))


def _rows(shape):
  return pl.BlockSpec(shape, lambda i: (i, 0))


def _fusion_body(v, a, t, wv, wa, wt, b, o):
  o[...] = (jnp.dot(v[...], wv[...], preferred_element_type=jnp.float32)
            + jnp.dot(a[...], wa[...], preferred_element_type=jnp.float32)
            + jnp.dot(t[...], wt[...], preferred_element_type=jnp.float32)
            + b[...])


def _tc_fusion(video, audio, title, wv, wa, wt, bf):
  return pl.pallas_call(
      _fusion_body,
      grid=(_ITEM // _BM,),
      in_specs=[_rows((_BM, 64)), _rows((_BM, 64)), _rows((_BM, 32)),
                _full((64, 64)), _full((64, 64)), _full((32, 64)),
                _full((1, 64))],
      out_specs=_rows((_BM, 64)),
      out_shape=jax.ShapeDtypeStruct((_ITEM, 64), jnp.float32),
  )(video, audio, title, wv, wa, wt, bf)


def _sage_body(agg, deg, x, wl, wr, b, h_ref, s_ref, q_ref):
  mean = agg[...] / jnp.maximum(deg[...], 1.0)
  h = (jnp.dot(mean, wl[...], preferred_element_type=jnp.float32)
       + jnp.dot(x[...], wr[...], preferred_element_type=jnp.float32)
       + b[...])
  h_ref[...] = h

  @pl.when(pl.program_id(0) == 0)
  def _():
    s_ref[...] = jnp.zeros_like(s_ref)
    q_ref[...] = jnp.zeros_like(q_ref)
  s_ref[...] += jnp.sum(h, axis=0, keepdims=True)
  q_ref[...] += jnp.sum(h * h, axis=0, keepdims=True)


def _tc_sage(agg, deg, x, wl, wr, b, width):
  return pl.pallas_call(
      _sage_body,
      grid=(_N // _BM,),
      in_specs=[_rows((_BM, _D)), _rows((_BM, 1)), _rows((_BM, _D)),
                _full((_D, width)), _full((_D, width)), _full((1, width))],
      out_specs=[_rows((_BM, width)), _full((1, width)), _full((1, width))],
      out_shape=[jax.ShapeDtypeStruct((_N, width), jnp.float32),
                 jax.ShapeDtypeStruct((1, width), jnp.float32),
                 jax.ShapeDtypeStruct((1, width), jnp.float32)],
  )(agg, deg, x, wl, wr, b)


def _mid_body(h, sc, sh, wl, wr, b, yw_ref, yr_ref):
  y = h[...] * sc[...] + sh[...]
  y = jnp.where(y > 0, y, 0.2 * y)
  yw_ref[...] = jnp.dot(y, wl[...], preferred_element_type=jnp.float32)
  yr_ref[...] = jnp.dot(y, wr[...], preferred_element_type=jnp.float32) + b[...]


def _tc_mid(h, scale, shift, w1l, w1r, b1):
  return pl.pallas_call(
      _mid_body,
      grid=(_N // _BM,),
      in_specs=[_rows((_BM, _HID)), _full((1, _HID)), _full((1, _HID)),
                _full((_HID, _D)), _full((_HID, _D)), _full((1, _D))],
      out_specs=[_rows((_BM, _D)), _rows((_BM, _D))],
      out_shape=[jax.ShapeDtypeStruct((_N, _D), jnp.float32),
                 jax.ShapeDtypeStruct((_N, _D), jnp.float32)],
  )(h, scale, shift, w1l, w1r, b1)


def _l1_body(agg, deg, yr, h_ref, s_ref, q_ref):
  h = agg[...] / jnp.maximum(deg[...], 1.0) + yr[...]
  h_ref[...] = h

  @pl.when(pl.program_id(0) == 0)
  def _():
    s_ref[...] = jnp.zeros_like(s_ref)
    q_ref[...] = jnp.zeros_like(q_ref)
  s_ref[...] += jnp.sum(h, axis=0, keepdims=True)
  q_ref[...] += jnp.sum(h * h, axis=0, keepdims=True)


def _tc_l1(agg, deg, yr):
  return pl.pallas_call(
      _l1_body,
      grid=(_N // _BM,),
      in_specs=[_rows((_BM, _D)), _rows((_BM, 1)), _rows((_BM, _D))],
      out_specs=[_rows((_BM, _D)), _full((1, _D)), _full((1, _D))],
      out_shape=[jax.ShapeDtypeStruct((_N, _D), jnp.float32),
                 jax.ShapeDtypeStruct((1, _D), jnp.float32),
                 jax.ShapeDtypeStruct((1, _D), jnp.float32)],
  )(agg, deg, yr)


def _final_body(x, h, sc, sh, o_ref):
  o_ref[...] = x[...] + h[...] * sc[...] + sh[...]


def _tc_final(x, h1, scale, shift):
  return pl.pallas_call(
      _final_body,
      grid=(_N // _BM,),
      in_specs=[_rows((_BM, _D)), _rows((_BM, _D)), _full((1, _D)),
                _full((1, _D))],
      out_specs=_rows((_BM, _D)),
      out_shape=jax.ShapeDtypeStruct((_N, _D), jnp.float32),
  )(x, h1, scale, shift)


def _bn_coeffs(ssum, ssq, g, be):
  m = ssum[0] / _N
  var = ssq[0] / _N - m * m
  scale = g / jnp.sqrt(var + 1e-5)
  shift = be - m * scale
  return scale[None, :], shift[None, :]


def kernel(video, audio, title, edge_index, user_emb, Wf, bf,
           W0l, W0r, b0, g0, be0, W1l, W1r, b1, g1, be1):
  src = edge_index[0]
  dst = edge_index[1]

  item_repr = _tc_fusion(video, audio, title,
                         Wf[:64], Wf[64:128], Wf[128:160], bf[None, :])
  all_emb = jnp.concatenate([user_emb, item_repr], axis=0)

  agg0_p, deg_p = _make_segsum(True)(all_emb, src, dst)
  agg0 = _unpad(agg0_p)
  deg = _unpad(deg_p)[:, None]

  h, ssum0, ssq0 = _tc_sage(agg0, deg, all_emb, W0l, W0r, b0[None, :], _HID)
  scale0, shift0 = _bn_coeffs(ssum0, ssq0, g0, be0)

  yw, yr = _tc_mid(h, scale0, shift0, W1l, W1r, b1[None, :])

  (agg1_p,) = _make_segsum(False)(yw, src, dst)
  agg1 = _unpad(agg1_p)

  h1, ssum1, ssq1 = _tc_l1(agg1, deg, yr)
  scale1, shift1 = _bn_coeffs(ssum1, ssq1, g1, be1)

  return _tc_final(all_emb, h1, scale1, shift1)


# speedup vs baseline: 6.9996x; 1.7843x over previous
import functools

import jax
import jax.numpy as jnp
from jax import lax
from jax.experimental import pallas as pl
from jax.experimental.pallas import tpu as pltpu
from jax.experimental.pallas import tpu_sc as plsc

_USER = 10000
_ITEM = 40000
_N = _USER + _ITEM
_E = 800000
_D = 64
_HID = 128

_NC = 2
_NS = 16
_HALF = _N // _NC
_DUMP_MASK = 1023
_ACC_ROWS = 26112
_RPT = _ACC_ROWS // _NS
_EB = 128
_NBLK = _E // _EB
_BQ, _BR = _NBLK // _NS, _NBLK % _NS


@functools.lru_cache(maxsize=None)
def _make_segsum(with_deg: bool):
  mesh = plsc.VectorSubcoreMesh(core_axis_name="c", subcore_axis_name="s",
                                num_cores=_NC, num_subcores=_NS)
  out_type = [jax.ShapeDtypeStruct((_NC * _ACC_ROWS, _D), jnp.float32)]
  scratch = [
      pltpu.VMEM((2, _EB), jnp.int32),
      pltpu.VMEM((2, _EB), jnp.int32),
      pltpu.VMEM((2, _EB, _D), jnp.float32),
      pltpu.VMEM((64, _D), jnp.float32),
      pltpu.VMEM_SHARED((_ACC_ROWS, _D), jnp.float32),
      pltpu.SemaphoreType.DMA((2,)),
      pltpu.SemaphoreType.DMA((2,)),
      pltpu.SemaphoreType.DMA((2,)),
  ]
  if with_deg:
    out_type.append(jax.ShapeDtypeStruct((_NC * _ACC_ROWS,), jnp.float32))
    scratch += [
        pltpu.VMEM((_EB,), jnp.float32),
        pltpu.VMEM((_RPT,), jnp.float32),
        pltpu.VMEM_SHARED((_ACC_ROWS,), jnp.float32),
        pltpu.SemaphoreType.DMA((2,)),
    ]

  @functools.partial(
      pl.kernel, mesh=mesh, out_type=tuple(out_type), scratch_types=scratch,
      compiler_params=pltpu.CompilerParams(use_tc_tiling_on_sc=False))
  def segsum(x_hbm, src_hbm, dst_hbm, *refs):
    if with_deg:
      (agg_out, deg_out, src_st, dst_st, rows_v, zrow_v, acc_sh,
       sem_st, sem_ga, sem_sc, ones_v, zdeg_v, deg_sh, sem_dg) = refs
    else:
      (agg_out, src_st, dst_st, rows_v, zrow_v, acc_sh,
       sem_st, sem_ga, sem_sc) = refs
    c = lax.axis_index("c")
    s = lax.axis_index("s")
    lo = c * _HALF
    hi = lo + _HALF
    row0 = s * _RPT
    z16 = jnp.zeros((16,), jnp.float32)
    lane = lax.iota(jnp.int32, 16)

    def zrow_body(r, _):
      for j in range(_D // 16):
        zrow_v[r, pl.ds(j * 16, 16)] = z16
      return 0
    lax.fori_loop(0, 64, zrow_body, 0)
    for k in range(_RPT // 64):
      pltpu.sync_copy(zrow_v, acc_sh.at[pl.ds(row0 + k * 64, 64)])
    rem = _RPT % 64
    if rem:
      pltpu.sync_copy(zrow_v.at[pl.ds(0, rem)],
                      acc_sh.at[pl.ds(row0 + _RPT - rem, rem)])
    if with_deg:
      def zdeg_body(r, _):
        zdeg_v[pl.ds(r * 16, 16)] = z16
        return 0
      lax.fori_loop(0, _RPT // 16, zdeg_body, 0)

      def ones_body(r, _):
        ones_v[pl.ds(r * 16, 16)] = z16 + 1.0
        return 0
      lax.fori_loop(0, _EB // 16, ones_body, 0)
      pltpu.sync_copy(zdeg_v, deg_sh.at[pl.ds(row0, _RPT)])
    plsc.subcore_barrier()

    blk0 = s * _BQ + jnp.minimum(s, _BR)
    nblk = _BQ + jnp.where(s < _BR, 1, 0)

    def start_stage(i, slot):
      e0 = (blk0 + i) * _EB
      pltpu.async_copy(src_hbm.at[pl.ds(e0, _EB)], src_st.at[slot],
                       sem_st.at[slot])
      pltpu.async_copy(dst_hbm.at[pl.ds(e0, _EB)], dst_st.at[slot],
                       sem_st.at[slot])

    start_stage(0, 0)

    def body(i, _):
      slot = lax.rem(i, 2)
      nslot = 1 - slot
      e0 = (blk0 + i) * _EB
      pltpu.make_async_copy(src_hbm.at[pl.ds(e0, _EB)], src_st.at[slot],
                            sem_st.at[slot]).wait()
      pltpu.make_async_copy(dst_hbm.at[pl.ds(e0, _EB)], dst_st.at[slot],
                            sem_st.at[slot]).wait()
      pltpu.async_copy(x_hbm.at[src_st.at[slot]], rows_v.at[slot],
                       sem_ga.at[slot])

      @pl.when(i >= 1)
      def _():
        pltpu.make_async_copy(rows_v.at[nslot], acc_sh.at[dst_st.at[nslot]],
                              sem_sc.at[nslot]).wait()
        if with_deg:
          pltpu.make_async_copy(ones_v, deg_sh.at[dst_st.at[nslot]],
                                sem_dg.at[nslot]).wait()

      @pl.when(i + 1 < nblk)
      def _():
        start_stage(i + 1, nslot)

      def remap(j, _):
        d = dst_st[slot, pl.ds(j * 16, 16)]
        mine = (d >= lo) & (d < hi)
        dump = _HALF + ((j * 16 + lane) & _DUMP_MASK)
        dst_st[slot, pl.ds(j * 16, 16)] = jnp.where(mine, d - lo, dump)
        return 0
      lax.fori_loop(0, _EB // 16, remap, 0)

      pltpu.make_async_copy(x_hbm.at[src_st.at[slot]], rows_v.at[slot],
                            sem_ga.at[slot]).wait()
      pltpu.async_copy(rows_v.at[slot], acc_sh.at[dst_st.at[slot]],
                       sem_sc.at[slot], add=True)
      if with_deg:
        pltpu.async_copy(ones_v, deg_sh.at[dst_st.at[slot]],
                         sem_dg.at[slot], add=True)
      return 0
    lax.fori_loop(0, nblk, body, 0)

    lslot = lax.rem(nblk - 1, 2)
    pltpu.make_async_copy(rows_v.at[lslot], acc_sh.at[dst_st.at[lslot]],
                          sem_sc.at[lslot]).wait()
    if with_deg:
      pltpu.make_async_copy(ones_v, deg_sh.at[dst_st.at[lslot]],
                            sem_dg.at[lslot]).wait()
    plsc.subcore_barrier()

    o0 = c * _ACC_ROWS + row0
    pltpu.sync_copy(acc_sh.at[pl.ds(row0, _RPT)], agg_out.at[pl.ds(o0, _RPT)])
    if with_deg:
      pltpu.sync_copy(deg_sh.at[pl.ds(row0, _RPT)], deg_out.at[pl.ds(o0, _RPT)])

  return segsum


def _unpad(a):
  return jnp.concatenate([a[:_HALF], a[_ACC_ROWS:_ACC_ROWS + _HALF]], axis=0)


_BM = 2000


def _full(shape):
  return pl.BlockSpec(shape, lambda i: (0, 0))


def _rows(shape):
  return pl.BlockSpec(shape, lambda i: (i, 0))


def _fusion_body(v, a, t, wv, wa, wt, b, o):
  o[...] = (jnp.dot(v[...], wv[...], preferred_element_type=jnp.float32)
            + jnp.dot(a[...], wa[...], preferred_element_type=jnp.float32)
            + jnp.dot(t[...], wt[...], preferred_element_type=jnp.float32)
            + b[...])


def _tc_fusion(video, audio, title, wv, wa, wt, bf):
  return pl.pallas_call(
      _fusion_body,
      grid=(_ITEM // _BM,),
      in_specs=[_rows((_BM, 64)), _rows((_BM, 64)), _rows((_BM, 32)),
                _full((64, 64)), _full((64, 64)), _full((32, 64)),
                _full((1, 64))],
      out_specs=_rows((_BM, 64)),
      out_shape=jax.ShapeDtypeStruct((_ITEM, 64), jnp.float32),
  )(video, audio, title, wv, wa, wt, bf)


def _sage_body(agg, deg, x, wl, wr, b, h_ref, s_ref, q_ref):
  mean = agg[...] / jnp.maximum(deg[...], 1.0)
  h = (jnp.dot(mean, wl[...], preferred_element_type=jnp.float32)
       + jnp.dot(x[...], wr[...], preferred_element_type=jnp.float32)
       + b[...])
  h_ref[...] = h

  @pl.when(pl.program_id(0) == 0)
  def _():
    s_ref[...] = jnp.zeros_like(s_ref)
    q_ref[...] = jnp.zeros_like(q_ref)
  s_ref[...] += jnp.sum(h, axis=0, keepdims=True)
  q_ref[...] += jnp.sum(h * h, axis=0, keepdims=True)


def _tc_sage(agg, deg, x, wl, wr, b, width):
  return pl.pallas_call(
      _sage_body,
      grid=(_N // _BM,),
      in_specs=[_rows((_BM, _D)), _rows((_BM, 1)), _rows((_BM, _D)),
                _full((_D, width)), _full((_D, width)), _full((1, width))],
      out_specs=[_rows((_BM, width)), _full((1, width)), _full((1, width))],
      out_shape=[jax.ShapeDtypeStruct((_N, width), jnp.float32),
                 jax.ShapeDtypeStruct((1, width), jnp.float32),
                 jax.ShapeDtypeStruct((1, width), jnp.float32)],
  )(agg, deg, x, wl, wr, b)


def _mid_body(h, sc, sh, wl, wr, b, yw_ref, yr_ref):
  y = h[...] * sc[...] + sh[...]
  y = jnp.where(y > 0, y, 0.2 * y)
  yw_ref[...] = jnp.dot(y, wl[...], preferred_element_type=jnp.float32)
  yr_ref[...] = jnp.dot(y, wr[...], preferred_element_type=jnp.float32) + b[...]


def _tc_mid(h, scale, shift, w1l, w1r, b1):
  return pl.pallas_call(
      _mid_body,
      grid=(_N // _BM,),
      in_specs=[_rows((_BM, _HID)), _full((1, _HID)), _full((1, _HID)),
                _full((_HID, _D)), _full((_HID, _D)), _full((1, _D))],
      out_specs=[_rows((_BM, _D)), _rows((_BM, _D))],
      out_shape=[jax.ShapeDtypeStruct((_N, _D), jnp.float32),
                 jax.ShapeDtypeStruct((_N, _D), jnp.float32)],
  )(h, scale, shift, w1l, w1r, b1)


def _l1_body(agg, deg, yr, h_ref, s_ref, q_ref):
  h = agg[...] / jnp.maximum(deg[...], 1.0) + yr[...]
  h_ref[...] = h

  @pl.when(pl.program_id(0) == 0)
  def _():
    s_ref[...] = jnp.zeros_like(s_ref)
    q_ref[...] = jnp.zeros_like(q_ref)
  s_ref[...] += jnp.sum(h, axis=0, keepdims=True)
  q_ref[...] += jnp.sum(h * h, axis=0, keepdims=True)


def _tc_l1(agg, deg, yr):
  return pl.pallas_call(
      _l1_body,
      grid=(_N // _BM,),
      in_specs=[_rows((_BM, _D)), _rows((_BM, 1)), _rows((_BM, _D))],
      out_specs=[_rows((_BM, _D)), _full((1, _D)), _full((1, _D))],
      out_shape=[jax.ShapeDtypeStruct((_N, _D), jnp.float32),
                 jax.ShapeDtypeStruct((1, _D), jnp.float32),
                 jax.ShapeDtypeStruct((1, _D), jnp.float32)],
  )(agg, deg, yr)


def _final_body(x, h, sc, sh, o_ref):
  o_ref[...] = x[...] + h[...] * sc[...] + sh[...]


def _tc_final(x, h1, scale, shift):
  return pl.pallas_call(
      _final_body,
      grid=(_N // _BM,),
      in_specs=[_rows((_BM, _D)), _rows((_BM, _D)), _full((1, _D)),
                _full((1, _D))],
      out_specs=_rows((_BM, _D)),
      out_shape=jax.ShapeDtypeStruct((_N, _D), jnp.float32),
  )(x, h1, scale, shift)


def _bn_coeffs(ssum, ssq, g, be):
  m = ssum[0] / _N
  var = ssq[0] / _N - m * m
  scale = g / jnp.sqrt(var + 1e-5)
  shift = be - m * scale
  return scale[None, :], shift[None, :]


def kernel(video, audio, title, edge_index, user_emb, Wf, bf,
           W0l, W0r, b0, g0, be0, W1l, W1r, b1, g1, be1):
  src = edge_index[0]
  dst = edge_index[1]

  item_repr = _tc_fusion(video, audio, title,
                         Wf[:64], Wf[64:128], Wf[128:160], bf[None, :])
  all_emb = jnp.concatenate([user_emb, item_repr], axis=0)

  agg0_p, deg_p = _make_segsum(True)(all_emb, src, dst)
  agg0 = _unpad(agg0_p)
  deg = _unpad(deg_p)[:, None]

  h, ssum0, ssq0 = _tc_sage(agg0, deg, all_emb, W0l, W0r, b0[None, :], _HID)
  scale0, shift0 = _bn_coeffs(ssum0, ssq0, g0, be0)

  yw, yr = _tc_mid(h, scale0, shift0, W1l, W1r, b1[None, :])

  (agg1_p,) = _make_segsum(False)(yw, src, dst)
  agg1 = _unpad(agg1_p)

  h1, ssum1, ssq1 = _tc_l1(agg1, deg, yr)
  scale1, shift1 = _bn_coeffs(ssum1, ssq1, g1, be1)

  return _tc_final(all_emb, h1, scale1, shift1)
